# baseline (speedup 1.0000x reference)
import functools

import jax
import jax.numpy as jnp
from jax import lax
from jax.experimental import pallas as pl
from jax.experimental.pallas import tpu as pltpu
from jax.experimental.pallas import tpu_sc as plsc

H = W = D = 128
V = H * W * D
B = 4
N = B * V
NW = 32
TAIL = 16520
P = N // NW
C = 2048
CH = C // 128
NCHUNK = P // C
CW = 4096
NBCHUNK = P // CW


def _bf(t):
    return lax.convert_element_type(
        lax.convert_element_type(t, jnp.bfloat16), jnp.float32)


def _coord(tf_ref, b, gxb, gyb, gzb, row):
    p0 = _bf(tf_ref[b, row * 3]) * gxb
    p1 = _bf(tf_ref[b, row * 3 + 1]) * gyb
    p2 = _bf(tf_ref[b, row * 3 + 2]) * gzb
    x_s = ((p0 + p1) + p2) + tf_ref[b, 9 + row]
    return 0.5 * (x_s + 1.0) * jnp.float32(127.0)


def _idx_body(tf_ref, idx_ref):
    b = pl.program_id(0)
    i = pl.program_id(1)
    scale = jnp.float32(2.0 / 127.0)
    gx = _bf(i.astype(jnp.float32) * scale - 1.0)
    gy = _bf(lax.broadcasted_iota(jnp.int32, (H, D), 0).astype(jnp.float32) * scale - 1.0)
    gz = _bf(lax.broadcasted_iota(jnp.int32, (H, D), 1).astype(jnp.float32) * scale - 1.0)
    x = _coord(tf_ref, b, gx, gy, gz, 0)
    y = _coord(tf_ref, b, gx, gy, gz, 1)
    z = _coord(tf_ref, b, gx, gy, gz, 2)
    x0 = jnp.clip(jnp.floor(x).astype(jnp.int32), 0, 127)
    y0 = jnp.clip(jnp.floor(y).astype(jnp.int32), 0, 127)
    z0 = jnp.clip(jnp.floor(z).astype(jnp.int32), 0, 127)
    idx_ref[0, 0] = ((b * H + x0) * W + y0) * D + z0


def _indices(Tform):
    return pl.pallas_call(
        _idx_body,
        grid=(B, W),
        in_specs=[pl.BlockSpec(memory_space=pltpu.SMEM)],
        out_specs=pl.BlockSpec((1, 1, H, D), lambda b, i: (b, i, 0, 0)),
        out_shape=jax.ShapeDtypeStruct((B, H, W, D), jnp.int32),
    )(Tform)


def _oct_build_body(flat_hbm, oct_hbm, win0, win1, out0, out1,
                    si0, si1, so0, so1):
    wid = lax.axis_index("s") * 2 + lax.axis_index("c")
    base0 = wid * P
    lanes = lax.broadcasted_iota(jnp.int32, (16,), 0)
    pat = ((lanes >> 3) + (lanes & 1) + ((lanes >> 1) & 1) * D
           + ((lanes >> 2) & 1) * (W * D))
    wins = (win0, win1)
    outs = (out0, out1)
    sis = (si0, si1)
    sos = (so0, so1)

    def start_in(ci, buf):
        base = pl.multiple_of(base0 + ci * CW, CW)
        pltpu.async_copy(flat_hbm.at[pl.ds(base, CW + TAIL)], wins[buf],
                         sis[buf])

    def wait_in(buf):
        pltpu.make_async_copy(flat_hbm.at[pl.ds(0, CW + TAIL)], wins[buf],
                              sis[buf]).wait()

    def wait_out(buf):
        pltpu.make_async_copy(flat_hbm.at[pl.ds(0, CW * 8)], outs[buf],
                              sos[buf]).wait()

    start_in(0, 0)

    def chunk_body(ci, carry):
        b = lax.rem(ci, 2)

        def with_buf(bufi):
            def inner():
                wait_in(bufi)

                @pl.when(ci + 1 < NBCHUNK)
                def _():
                    start_in(ci + 1, 1 - bufi)

                @pl.when(ci >= 2)
                def _():
                    wait_out(bufi)

                def grp(g, c2):
                    g8 = g * 8
                    for u in range(8):
                        outs[bufi][pl.ds((g8 + u) * 16, 16)] = \
                            plsc.load_gather(wins[bufi], [pat + (g8 + u) * 2])
                    return c2

                lax.fori_loop(0, CW * 8 // 128, grp, 0)
                base = pl.multiple_of(base0 + ci * CW, CW)
                pltpu.async_copy(outs[bufi],
                                 oct_hbm.at[pl.ds(base * 8, CW * 8)],
                                 sos[bufi])
            return inner

        pl.when(b == 0)(with_buf(0))
        pl.when(b == 1)(with_buf(1))
        return carry

    lax.fori_loop(0, NBCHUNK, chunk_body, 0)
    wait_out(0)
    wait_out(1)


def _sc_gather_body(oct_hbm, idx_hbm, vals_hbm, idx0, idx1, rows0, rows1,
                    out0, out1, si0, si1, sg0, sg1, so0, so1,
                    npts=N):
    nchunk = npts // NW // C
    wid = lax.axis_index("s") * 2 + lax.axis_index("c")
    base0 = wid * (npts // NW)
    lanes = lax.broadcasted_iota(jnp.int32, (16,), 0)
    idxs = (idx0, idx1)
    rows = (rows0, rows1)
    outs = (out0, out1)
    sis = (si0, si1)
    sgs = (sg0, sg1)
    sos = (so0, so1)

    def start_idx(ci, buf):
        base = pl.multiple_of(base0 + ci * C, C)
        rbase = pl.multiple_of(base // 128, CH)
        pltpu.async_copy(idx_hbm.at[pl.ds(rbase, CH)], idxs[buf], sis[buf])

    def wait_idx(buf):
        pltpu.make_async_copy(idx_hbm.at[pl.ds(0, CH)], idxs[buf],
                              sis[buf]).wait()

    def fire_gathers(buf):
        def fire(r, c2):
            pltpu.async_copy(oct_hbm.at[idxs[buf].at[r]],
                             rows[buf].at[pl.ds(r * 128, 128)], sgs[buf])
            return c2

        lax.fori_loop(0, CH, fire, 0)

    def drain_gathers(buf):
        pltpu.make_async_copy(oct_hbm.at[pl.ds(0, C)], rows[buf],
                              sgs[buf]).wait()

    def wait_out(buf):
        pltpu.make_async_copy(vals_hbm.at[pl.ds(0, C * 8)], outs[buf],
                              sos[buf]).wait()

    start_idx(0, 0)
    wait_idx(0)
    fire_gathers(0)
    start_idx(1, 1)

    def chunk_body(ci, carry):
        b = lax.rem(ci, 2)

        def with_buf(bufi):
            def inner():
                drain_gathers(bufi)

                @pl.when(ci + 1 < nchunk)
                def _():
                    wait_idx(1 - bufi)
                    fire_gathers(1 - bufi)

                @pl.when(ci + 2 < nchunk)
                def _():
                    start_idx(ci + 2, bufi)

                @pl.when(ci >= 2)
                def _():
                    wait_out(bufi)

                def grp(t8, c2):
                    for u in range(8):
                        t = t8 * 8 + u
                        pid = (t >> 6) * 128 + (t & 7) * 16 + lanes
                        crn = jnp.broadcast_to((t >> 3) & 7, (16,))
                        outs[bufi][pl.ds(t * 16, 16)] = plsc.load_gather(
                            rows[bufi], [pid, crn])
                    return c2

                lax.fori_loop(0, C * 8 // 128, grp, 0)
                base = pl.multiple_of(base0 + ci * C, C)
                pltpu.async_copy(outs[bufi],
                                 vals_hbm.at[pl.ds(base * 8, C * 8)],
                                 sos[bufi])
            return inner

        pl.when(b == 0)(with_buf(0))
        pl.when(b == 1)(with_buf(1))
        return carry

    lax.fori_loop(0, nchunk, chunk_body, 0)
    wait_out(0)
    wait_out(1)


@functools.cache
def _sc_kernels():
    mesh = plsc.VectorSubcoreMesh(core_axis_name="c", subcore_axis_name="s",
                                  num_cores=2, num_subcores=16)
    params = pltpu.CompilerParams(use_tc_tiling_on_sc=False,
                                  needs_layout_passes=False)
    build = pl.kernel(
        _oct_build_body,
        out_type=jax.ShapeDtypeStruct((N * 8,), jnp.float32),
        mesh=mesh,
        scratch_types=[
            pltpu.VMEM((CW + TAIL,), jnp.float32),
            pltpu.VMEM((CW + TAIL,), jnp.float32),
            pltpu.VMEM((CW * 8,), jnp.float32),
            pltpu.VMEM((CW * 8,), jnp.float32),
            pltpu.SemaphoreType.DMA,
            pltpu.SemaphoreType.DMA,
            pltpu.SemaphoreType.DMA,
            pltpu.SemaphoreType.DMA,
        ],
        compiler_params=params,
    )
    npts = N // 2
    gather = pl.kernel(
        functools.partial(_sc_gather_body, npts=npts),
        out_type=jax.ShapeDtypeStruct((npts * 8,), jnp.float32),
        mesh=mesh,
        scratch_types=[
            pltpu.VMEM((CH, 128), jnp.int32),
            pltpu.VMEM((CH, 128), jnp.int32),
            pltpu.VMEM((C, 8), jnp.float32),
            pltpu.VMEM((C, 8), jnp.float32),
            pltpu.VMEM((C * 8,), jnp.float32),
            pltpu.VMEM((C * 8,), jnp.float32),
            pltpu.SemaphoreType.DMA,
            pltpu.SemaphoreType.DMA,
            pltpu.SemaphoreType.DMA,
            pltpu.SemaphoreType.DMA,
            pltpu.SemaphoreType.DMA,
            pltpu.SemaphoreType.DMA,
        ],
        compiler_params=params,
    )
    return build, gather


def _combine_body(tf_ref, vals_ref, sum8_ref, out_ref, plane0=0):
    pid = plane0 + pl.program_id(0)
    b = pid // H
    i = pid % H
    scale = jnp.float32(2.0 / 127.0)
    gx = _bf(i.astype(jnp.float32) * scale - 1.0)
    r = lax.broadcasted_iota(jnp.int32, (1, 1024, 128), 1)
    k = lax.broadcasted_iota(jnp.int32, (1, 1024, 128), 2)
    j = r >> 3
    corner = r & 7
    gy = _bf(j.astype(jnp.float32) * scale - 1.0)
    gz = _bf(k.astype(jnp.float32) * scale - 1.0)
    x = _coord(tf_ref, b, gx, gy, gz, 0)
    y = _coord(tf_ref, b, gx, gy, gz, 1)
    z = _coord(tf_ref, b, gx, gy, gz, 2)
    fone = jnp.float32(1.0)
    fzero = jnp.float32(0.0)

    def axis_w(t, bit):
        f = t - jnp.floor(t)
        wsel = jnp.where(bit == 1, f, fone - f)
        lim = jnp.float32(128.0) - bit.astype(jnp.float32)
        return jnp.where((t >= 0.0) & (t < lim), wsel, fzero)

    wx = axis_w(x, lax.shift_right_logical(corner, 2) & 1)
    wy = axis_w(y, lax.shift_right_logical(corner, 1) & 1)
    wz = axis_w(z, corner & 1)
    prod = (vals_ref[...] * (wx * wy * wz)).reshape(1024, 128)
    s = lax.dot_general(sum8_ref[...], prod, (((1,), (0,)), ((), ())),
                        precision=lax.Precision.HIGHEST)
    out_ref[...] = s.reshape(1, W, D)


def _combine(Tform, vals, plane0, nplanes):
    sum8 = jnp.repeat(jnp.eye(W, dtype=jnp.float32), 8, axis=1)
    return pl.pallas_call(
        functools.partial(_combine_body, plane0=plane0),
        grid=(nplanes,),
        in_specs=[
            pl.BlockSpec(memory_space=pltpu.SMEM),
            pl.BlockSpec((1, 1024, 128), lambda p: (p, 0, 0)),
            pl.BlockSpec((W, 1024), lambda p: (0, 0)),
        ],
        out_specs=pl.BlockSpec((1, W, D), lambda p: (p, 0, 0)),
        out_shape=jax.ShapeDtypeStruct((nplanes, W, D), jnp.float32),
    )(Tform, vals.reshape(nplanes, 1024, 128), sum8)


def kernel(Img, Tform):
    idx = _indices(Tform)
    build, gather = _sc_kernels()
    flat_ext = jnp.concatenate(
        [Img.reshape(N), jnp.zeros((TAIL,), jnp.float32)])
    octt = build(flat_ext).reshape(N, 8)
    idx2 = idx.reshape(N // 128, 128)
    NH = N // 2
    NP = B * H // 2
    halves = []
    for h in range(2):
        vals = gather(octt, lax.slice_in_dim(idx2, h * (NH // 128),
                                             (h + 1) * (NH // 128)))
        halves.append(_combine(Tform, vals, h * NP, NP))
    out = jnp.concatenate(halves, axis=0)
    return out.reshape(B, H, W, D, 1)


# speedup vs baseline: 1.1008x; 1.1008x over previous
import functools

import jax
import jax.numpy as jnp
from jax import lax
from jax.experimental import pallas as pl
from jax.experimental.pallas import tpu as pltpu
from jax.experimental.pallas import tpu_sc as plsc

H = W = D = 128
V = H * W * D
B = 4
N = B * V
NW = 32
TAIL = 16520
P = N // NW
C = 2048
CH = C // 128
NCHUNK = P // C
CW = 4096
NBCHUNK = P // CW


def _bf(t):
    return lax.convert_element_type(
        lax.convert_element_type(t, jnp.bfloat16), jnp.float32)


def _coord(tf_ref, b, gxb, gyb, gzb, row):
    p0 = _bf(tf_ref[b, row * 3]) * gxb
    p1 = _bf(tf_ref[b, row * 3 + 1]) * gyb
    p2 = _bf(tf_ref[b, row * 3 + 2]) * gzb
    x_s = ((p0 + p1) + p2) + tf_ref[b, 9 + row]
    return 0.5 * (x_s + 1.0) * jnp.float32(127.0)


def _idx_body(tf_ref, idx_ref):
    b = pl.program_id(0)
    i = pl.program_id(1)
    scale = jnp.float32(2.0 / 127.0)
    gx = _bf(i.astype(jnp.float32) * scale - 1.0)
    gy = _bf(lax.broadcasted_iota(jnp.int32, (H, D), 0).astype(jnp.float32) * scale - 1.0)
    gz = _bf(lax.broadcasted_iota(jnp.int32, (H, D), 1).astype(jnp.float32) * scale - 1.0)
    x = _coord(tf_ref, b, gx, gy, gz, 0)
    y = _coord(tf_ref, b, gx, gy, gz, 1)
    z = _coord(tf_ref, b, gx, gy, gz, 2)
    x0 = jnp.clip(jnp.floor(x).astype(jnp.int32), 0, 127)
    y0 = jnp.clip(jnp.floor(y).astype(jnp.int32), 0, 127)
    z0 = jnp.clip(jnp.floor(z).astype(jnp.int32), 0, 127)
    idx_ref[0, 0] = ((b * H + x0) * W + y0) * D + z0


def _indices(Tform):
    return pl.pallas_call(
        _idx_body,
        grid=(B, W),
        in_specs=[pl.BlockSpec(memory_space=pltpu.SMEM)],
        out_specs=pl.BlockSpec((1, 1, H, D), lambda b, i: (b, i, 0, 0)),
        out_shape=jax.ShapeDtypeStruct((B, H, W, D), jnp.int32),
    )(Tform)


def _oct_build_body(flat_hbm, oct_hbm, win0, win1, out0, out1,
                    si0, si1, so0, so1):
    wid = lax.axis_index("s") * 2 + lax.axis_index("c")
    base0 = wid * P
    lanes = lax.broadcasted_iota(jnp.int32, (16,), 0)
    pat = ((lanes >> 3) + (lanes & 1) + ((lanes >> 1) & 1) * D
           + ((lanes >> 2) & 1) * (W * D))
    wins = (win0, win1)
    outs = (out0, out1)
    sis = (si0, si1)
    sos = (so0, so1)

    def start_in(ci, buf):
        base = pl.multiple_of(base0 + ci * CW, CW)
        pltpu.async_copy(flat_hbm.at[pl.ds(base, CW + TAIL)], wins[buf],
                         sis[buf])

    def wait_in(buf):
        pltpu.make_async_copy(flat_hbm.at[pl.ds(0, CW + TAIL)], wins[buf],
                              sis[buf]).wait()

    def wait_out(buf):
        pltpu.make_async_copy(flat_hbm.at[pl.ds(0, CW * 8)], outs[buf],
                              sos[buf]).wait()

    start_in(0, 0)

    def chunk_body(ci, carry):
        b = lax.rem(ci, 2)

        def with_buf(bufi):
            def inner():
                wait_in(bufi)

                @pl.when(ci + 1 < NBCHUNK)
                def _():
                    start_in(ci + 1, 1 - bufi)

                @pl.when(ci >= 2)
                def _():
                    wait_out(bufi)

                def grp(g, c2):
                    g8 = g * 8
                    for u in range(8):
                        outs[bufi][pl.ds((g8 + u) * 16, 16)] = \
                            plsc.load_gather(wins[bufi], [pat + (g8 + u) * 2])
                    return c2

                lax.fori_loop(0, CW * 8 // 128, grp, 0)
                base = pl.multiple_of(base0 + ci * CW, CW)
                pltpu.async_copy(outs[bufi],
                                 oct_hbm.at[pl.ds(base * 8, CW * 8)],
                                 sos[bufi])
            return inner

        pl.when(b == 0)(with_buf(0))
        pl.when(b == 1)(with_buf(1))
        return carry

    lax.fori_loop(0, NBCHUNK, chunk_body, 0)
    wait_out(0)
    wait_out(1)


def _sc_gather_body(oct_hbm, idx_hbm, vals_hbm, idx0, idx1, rows0, rows1,
                    out0, out1, si0, si1, sg0, sg1, so0, so1,
                    npts=N):
    nchunk = npts // NW // C
    wid = lax.axis_index("s") * 2 + lax.axis_index("c")
    base0 = wid * (npts // NW)
    lanes = lax.broadcasted_iota(jnp.int32, (16,), 0)
    idxs = (idx0, idx1)
    rows = (rows0, rows1)
    outs = (out0, out1)
    sis = (si0, si1)
    sgs = (sg0, sg1)
    sos = (so0, so1)

    def start_idx(ci, buf):
        base = pl.multiple_of(base0 + ci * C, C)
        rbase = pl.multiple_of(base // 128, CH)
        pltpu.async_copy(idx_hbm.at[pl.ds(rbase, CH)], idxs[buf], sis[buf])

    def wait_idx(buf):
        pltpu.make_async_copy(idx_hbm.at[pl.ds(0, CH)], idxs[buf],
                              sis[buf]).wait()

    def fire_gathers(buf):
        def fire(r, c2):
            pltpu.async_copy(oct_hbm.at[idxs[buf].at[r]],
                             rows[buf].at[pl.ds(r * 128, 128)], sgs[buf])
            return c2

        lax.fori_loop(0, CH, fire, 0)

    def drain_gathers(buf):
        pltpu.make_async_copy(oct_hbm.at[pl.ds(0, C)], rows[buf],
                              sgs[buf]).wait()

    def wait_out(buf):
        pltpu.make_async_copy(vals_hbm.at[pl.ds(0, C * 8)], outs[buf],
                              sos[buf]).wait()

    start_idx(0, 0)
    wait_idx(0)
    fire_gathers(0)
    start_idx(1, 1)

    def chunk_body(ci, carry):
        b = lax.rem(ci, 2)

        def with_buf(bufi):
            def inner():
                drain_gathers(bufi)

                @pl.when(ci + 1 < nchunk)
                def _():
                    wait_idx(1 - bufi)
                    fire_gathers(1 - bufi)

                @pl.when(ci + 2 < nchunk)
                def _():
                    start_idx(ci + 2, bufi)

                @pl.when(ci >= 2)
                def _():
                    wait_out(bufi)

                def grp(t8, c2):
                    for u in range(8):
                        t = t8 * 8 + u
                        pid = (t >> 6) * 128 + (t & 7) * 16 + lanes
                        crn = jnp.broadcast_to((t >> 3) & 7, (16,))
                        outs[bufi][pl.ds(t * 16, 16)] = plsc.load_gather(
                            rows[bufi], [pid, crn])
                    return c2

                lax.fori_loop(0, C * 8 // 128, grp, 0)
                base = pl.multiple_of(base0 + ci * C, C)
                pltpu.async_copy(outs[bufi],
                                 vals_hbm.at[pl.ds(base * 8, C * 8)],
                                 sos[bufi])
            return inner

        pl.when(b == 0)(with_buf(0))
        pl.when(b == 1)(with_buf(1))
        return carry

    lax.fori_loop(0, nchunk, chunk_body, 0)
    wait_out(0)
    wait_out(1)


@functools.cache
def _sc_kernels():
    mesh = plsc.VectorSubcoreMesh(core_axis_name="c", subcore_axis_name="s",
                                  num_cores=2, num_subcores=16)
    params = pltpu.CompilerParams(use_tc_tiling_on_sc=False,
                                  needs_layout_passes=False)
    build = pl.kernel(
        _oct_build_body,
        out_type=jax.ShapeDtypeStruct((N * 8,), jnp.float32),
        mesh=mesh,
        scratch_types=[
            pltpu.VMEM((CW + TAIL,), jnp.float32),
            pltpu.VMEM((CW + TAIL,), jnp.float32),
            pltpu.VMEM((CW * 8,), jnp.float32),
            pltpu.VMEM((CW * 8,), jnp.float32),
            pltpu.SemaphoreType.DMA,
            pltpu.SemaphoreType.DMA,
            pltpu.SemaphoreType.DMA,
            pltpu.SemaphoreType.DMA,
        ],
        compiler_params=params,
    )
    npts = N
    gather = pl.kernel(
        functools.partial(_sc_gather_body, npts=npts),
        out_type=jax.ShapeDtypeStruct((npts * 8,), jnp.float32),
        mesh=mesh,
        scratch_types=[
            pltpu.VMEM((CH, 128), jnp.int32),
            pltpu.VMEM((CH, 128), jnp.int32),
            pltpu.VMEM((C, 8), jnp.float32),
            pltpu.VMEM((C, 8), jnp.float32),
            pltpu.VMEM((C * 8,), jnp.float32),
            pltpu.VMEM((C * 8,), jnp.float32),
            pltpu.SemaphoreType.DMA,
            pltpu.SemaphoreType.DMA,
            pltpu.SemaphoreType.DMA,
            pltpu.SemaphoreType.DMA,
            pltpu.SemaphoreType.DMA,
            pltpu.SemaphoreType.DMA,
        ],
        compiler_params=params,
    )
    return build, gather


def _combine_body(tf_ref, vals_ref, sum8_ref, out_ref, plane0=0):
    pid = plane0 + pl.program_id(0)
    b = pid // H
    i = pid % H
    scale = jnp.float32(2.0 / 127.0)
    gx = _bf(i.astype(jnp.float32) * scale - 1.0)
    r = lax.broadcasted_iota(jnp.int32, (1, 1024, 128), 1)
    k = lax.broadcasted_iota(jnp.int32, (1, 1024, 128), 2)
    j = r >> 3
    corner = r & 7
    gy = _bf(j.astype(jnp.float32) * scale - 1.0)
    gz = _bf(k.astype(jnp.float32) * scale - 1.0)
    x = _coord(tf_ref, b, gx, gy, gz, 0)
    y = _coord(tf_ref, b, gx, gy, gz, 1)
    z = _coord(tf_ref, b, gx, gy, gz, 2)
    fone = jnp.float32(1.0)
    fzero = jnp.float32(0.0)

    def axis_w(t, bit):
        f = t - jnp.floor(t)
        wsel = jnp.where(bit == 1, f, fone - f)
        lim = jnp.float32(128.0) - bit.astype(jnp.float32)
        return jnp.where((t >= 0.0) & (t < lim), wsel, fzero)

    wx = axis_w(x, lax.shift_right_logical(corner, 2) & 1)
    wy = axis_w(y, lax.shift_right_logical(corner, 1) & 1)
    wz = axis_w(z, corner & 1)
    prod = (vals_ref[...] * (wx * wy * wz)).reshape(1024, 128)
    s = lax.dot_general(sum8_ref[...], prod, (((1,), (0,)), ((), ())),
                        precision=lax.Precision.HIGHEST)
    out_ref[...] = s.reshape(1, W, D)


def _combine(Tform, vals, plane0, nplanes):
    sum8 = jnp.repeat(jnp.eye(W, dtype=jnp.float32), 8, axis=1)
    return pl.pallas_call(
        functools.partial(_combine_body, plane0=plane0),
        grid=(nplanes,),
        in_specs=[
            pl.BlockSpec(memory_space=pltpu.SMEM),
            pl.BlockSpec((1, 1024, 128), lambda p: (p, 0, 0)),
            pl.BlockSpec((W, 1024), lambda p: (0, 0)),
        ],
        out_specs=pl.BlockSpec((1, W, D), lambda p: (p, 0, 0)),
        out_shape=jax.ShapeDtypeStruct((nplanes, W, D), jnp.float32),
    )(Tform, vals.reshape(nplanes, 1024, 128), sum8)


def kernel(Img, Tform):
    idx = _indices(Tform)
    build, gather = _sc_kernels()
    flat_ext = jnp.concatenate(
        [Img.reshape(N), jnp.zeros((TAIL,), jnp.float32)])
    octt = build(flat_ext).reshape(N, 8)
    vals = gather(octt, idx.reshape(N // 128, 128))
    out = _combine(Tform, vals, 0, B * H)
    return out.reshape(B, H, W, D, 1)
